# Initial kernel scaffold; baseline (speedup 1.0000x reference)
import jax
import jax.numpy as jnp
from jax.experimental import pallas as pl


def kernel(x, edge_index, batch_idx, W_pre0, b_pre0, W_pre1, b_pre1, W_mp0, b_mp0, W_mp1, b_mp1, W_mp2, b_mp2, W_mp3, b_mp3, W_post0, b_post0, W_post1, b_post1, W_out, b_out):
    raise NotImplementedError("write your pallas kernel here")


# speedup vs baseline: 18.5594x; 18.5594x over previous
import functools
import math

import jax
import jax.numpy as jnp
from jax import lax
from jax.experimental import pallas as pl
from jax.experimental.pallas import tpu as pltpu
from jax.experimental.pallas import tpu_sc as plsc

N = 10000
E = 320000
F = 128
H = 16
B = 16
INV = 1.0 / math.sqrt(1.0 + 1e-3)

NC = 2
NS = 16
NW = NC * NS
EPW = E // NW
CW = 80
NCH = EPW // CW
RPT = N // NS
NP = N // 8
PK = 8 * H


def _sigbn(v):
    return jax.nn.sigmoid(INV * v)


def _dot(a, b):
    return jnp.dot(a, b, preferred_element_type=jnp.float32)


def _seg_body(m_hbm, srcr, dstr, zeros_hbm, out_hbm,
              acc_sh, idxs_v, idxd_v, rows0, rows1, sem0, sem1):
    cid = lax.axis_index("c")
    sid = lax.axis_index("s")
    w = cid * NS + sid
    pltpu.sync_copy(zeros_hbm.at[pl.ds(sid * RPT, RPT)],
                    acc_sh.at[pl.ds(sid * RPT, RPT)])
    pltpu.sync_copy(srcr.at[w], idxs_v)
    pltpu.sync_copy(dstr.at[w], idxd_v)
    plsc.subcore_barrier()

    def _gather(j, buf, sem):
        return pltpu.make_async_copy(m_hbm.at[idxs_v.at[j]], buf, sem)

    def _scat(j, buf):
        pltpu.sync_copy(buf, acc_sh.at[idxd_v.at[j]], add=True)

    _gather(0, rows0, sem0).start()

    def body(k, carry):
        j = k * 2
        _gather(j + 1, rows1, sem1).start()
        _gather(j, rows0, sem0).wait()
        _scat(j, rows0)
        _gather(j + 2, rows0, sem0).start()
        _gather(j + 1, rows1, sem1).wait()
        _scat(j + 1, rows1)
        return carry

    lax.fori_loop(0, NCH // 2, body, 0)
    _gather(NCH - 1, rows0, sem0).wait()
    _scat(NCH - 1, rows0)
    plsc.subcore_barrier()
    pltpu.sync_copy(acc_sh.at[pl.ds(sid * RPT, RPT)],
                    out_hbm.at[cid, pl.ds(sid * RPT, RPT)])


_seg_call = functools.partial(
    pl.kernel,
    out_type=jax.ShapeDtypeStruct((NC, N, H), jnp.float32),
    mesh=plsc.VectorSubcoreMesh(core_axis_name="c", subcore_axis_name="s"),
    compiler_params=pltpu.CompilerParams(use_tc_tiling_on_sc=False),
    scratch_types=[
        pltpu.VMEM_SHARED((N, H), jnp.float32),
        pltpu.VMEM((NCH, CW), jnp.int32),
        pltpu.VMEM((NCH, CW), jnp.int32),
        pltpu.VMEM((CW, H), jnp.float32),
        pltpu.VMEM((CW, H), jnp.float32),
        pltpu.SemaphoreType.DMA,
        pltpu.SemaphoreType.DMA,
    ],
)(_seg_body)


def _full(shape):
    return pl.BlockSpec(shape, lambda: tuple(0 for _ in shape))


def _pre_body(x_ref, w0_ref, b0_ref, w1bd_ref, b1_ref, wmbd_ref, bm_ref,
              hp_ref, m_ref):
    x3 = x_ref[...].reshape(NP, 8, F)
    w0 = w0_ref[...]
    parts = [_dot(x3[:, a, :], w0) for a in range(8)]
    h = _sigbn(jnp.concatenate(parts, axis=1) + b0_ref[...])
    h = _sigbn(_dot(h, w1bd_ref[...]) + b1_ref[...])
    hp_ref[...] = h
    m_ref[...] = _dot(h, wmbd_ref[...]) + bm_ref[...]


def _pre_call(x, w0, b0t, w1bd, b1t, wmbd, bmt):
    return pl.pallas_call(
        _pre_body,
        grid=(),
        in_specs=[_full((N, F)), _full((F, H)), _full((1, PK)),
                  _full((PK, PK)), _full((1, PK)),
                  _full((PK, PK)), _full((1, PK))],
        out_specs=[_full((NP, PK)), _full((NP, PK))],
        out_shape=[jax.ShapeDtypeStruct((NP, PK), jnp.float32),
                   jax.ShapeDtypeStruct((NP, PK), jnp.float32)],
    )(x, w0, b0t, w1bd, b1t, wmbd, bmt)


def _make_mid_body(nz):
    def body(*refs):
        p_ref = refs[0]
        piece_refs = refs[1:1 + nz]
        w_refs = refs[1 + nz:2 + 2 * nz]
        b_ref = refs[2 + 2 * nz]
        z_ref, m_ref = refs[3 + 2 * nz], refs[4 + 2 * nz]
        z = _sigbn(p_ref[0] + p_ref[1])
        z_ref[...] = z
        m = _dot(z, w_refs[0][...]) + b_ref[...]
        for pr, wr in zip(piece_refs, w_refs[1:]):
            m = m + _dot(pr[...], wr[...])
        m_ref[...] = m
    return body


def _mid_call(p, pieces, wbd_slices, bt):
    nz = len(pieces)
    return pl.pallas_call(
        _make_mid_body(nz),
        grid=(),
        in_specs=([_full((NC, NP, PK))]
                  + [_full((NP, PK))] * nz
                  + [_full((PK, PK))] * (nz + 1)
                  + [_full((1, PK))]),
        out_specs=[_full((NP, PK)), _full((NP, PK))],
        out_shape=[jax.ShapeDtypeStruct((NP, PK), jnp.float32),
                   jax.ShapeDtypeStruct((NP, PK), jnp.float32)],
    )(p, *pieces, *wbd_slices, bt)


def _final_body(p_ref, z3_ref, z2_ref, z1_ref, hp_ref, bidx_ref,
                wp0_ref, bp0_ref, wp1_ref, bp1_ref, wo_ref, bo_ref, out_ref):
    z4 = _sigbn(p_ref[0] + p_ref[1])
    pieces = [z4, z3_ref[...], z2_ref[...], z1_ref[...], hp_ref[...]]
    bidx = bidx_ref[...]
    iota = lax.broadcasted_iota(jnp.int32, (B, NP), 0)
    g = jnp.zeros((B, 5 * H), jnp.float32)
    for a in range(8):
        oh = (bidx[a, :][None, :] == iota).astype(jnp.float32)
        cols = jnp.concatenate([pc[:, a * H:(a + 1) * H] for pc in pieces],
                               axis=1)
        g = g + _dot(oh, cols)
    g = _sigbn(_dot(g, wp0_ref[...]) + bp0_ref[...])
    g = _sigbn(_dot(g, wp1_ref[...]) + bp1_ref[...])
    out_ref[...] = jax.nn.sigmoid(_dot(g, wo_ref[...]) + bo_ref[...])


def _final_call(p, pieces, bidx_s, wp0, bp0, wp1, bp1, wo, bo):
    return pl.pallas_call(
        _final_body,
        grid=(),
        in_specs=([_full((NC, NP, PK))]
                  + [_full((NP, PK))] * 4
                  + [_full((8, NP))]
                  + [_full((5 * H, H)), _full((1, H)),
                     _full((H, H)), _full((1, H)),
                     _full((H, 1)), _full((1, 1))]),
        out_specs=_full((B, 1)),
        out_shape=jax.ShapeDtypeStruct((B, 1), jnp.float32),
    )(p, *pieces, bidx_s, wp0, bp0, wp1, bp1, wo, bo)


def kernel(x, edge_index, batch_idx, W_pre0, b_pre0, W_pre1, b_pre1,
           W_mp0, b_mp0, W_mp1, b_mp1, W_mp2, b_mp2, W_mp3, b_mp3,
           W_post0, b_post0, W_post1, b_post1, W_out, b_out):
    src_r = edge_index[0].reshape(NW, NCH, CW)
    dst_r = edge_index[1].reshape(NW, NCH, CW)
    zeros = jnp.zeros((N, H), jnp.float32)
    bidx_s = batch_idx.reshape(NP, 8).T

    def bd(wm):
        return jnp.kron(jnp.eye(8, dtype=jnp.float32), wm)

    def bt(b):
        return jnp.tile(b, 8).reshape(1, PK)

    def bd_slices(wm):
        return [bd(wm[k * H:(k + 1) * H]) for k in range(wm.shape[0] // H)]

    hp, m = _pre_call(x, W_pre0, bt(b_pre0), bd(W_pre1), bt(b_pre1),
                      bd(W_mp0), bt(b_mp0))
    pieces = [hp]
    for wm, bm in ((W_mp1, b_mp1), (W_mp2, b_mp2), (W_mp3, b_mp3)):
        p = _seg_call(m.reshape(N, H), src_r, dst_r, zeros)
        z, m = _mid_call(p.reshape(NC, NP, PK), pieces, bd_slices(wm), bt(bm))
        pieces.insert(0, z)
    p = _seg_call(m.reshape(N, H), src_r, dst_r, zeros)
    return _final_call(p.reshape(NC, NP, PK), pieces, bidx_s,
                       W_post0, b_post0.reshape(1, H),
                       W_post1, b_post1.reshape(1, H),
                       W_out, b_out.reshape(1, 1))


# speedup vs baseline: 29.3267x; 1.5802x over previous
import functools
import math

import jax
import jax.numpy as jnp
from jax import lax
from jax.experimental import pallas as pl
from jax.experimental.pallas import tpu as pltpu
from jax.experimental.pallas import tpu_sc as plsc

N = 10000
E = 320000
F = 128
H = 16
B = 16
INV = 1.0 / math.sqrt(1.0 + 1e-3)

NC = 2
NS = 16
NW = NC * NS
CW = 128
NCH = 79
EPW = NCH * CW
EPAD = NW * EPW
NA = N + 16
RPT = NA // NS
NP = N // 8
PK = 8 * H


def _sigbn(v):
    return jax.nn.sigmoid(INV * v)


def _dot(a, b):
    return jnp.dot(a, b, preferred_element_type=jnp.float32)


def _seg_body(m_hbm, srcr, dstr, zeros_hbm, out_hbm, acc_sh, idxs_v, idxd_v,
              r0, r1, r2, r3, g0, g1, g2, g3, s0, s1, s2, s3):
    rows = (r0, r1, r2, r3)
    gsem = (g0, g1, g2, g3)
    ssem = (s0, s1, s2, s3)
    cid = lax.axis_index("c")
    sid = lax.axis_index("s")
    w = cid * NS + sid
    pltpu.sync_copy(zeros_hbm.at[pl.ds(sid * RPT, RPT)],
                    acc_sh.at[pl.ds(sid * RPT, RPT)])
    pltpu.sync_copy(srcr.at[w], idxs_v)
    pltpu.sync_copy(dstr.at[w], idxd_v)
    plsc.subcore_barrier()

    def gst(j, b):
        pltpu.make_async_copy(m_hbm.at[idxs_v.at[j]], rows[b], gsem[b]).start()

    def gwt(b):
        pltpu.make_async_copy(m_hbm.at[idxs_v.at[0]], rows[b], gsem[b]).wait()

    def sst(j, b):
        pltpu.make_async_copy(rows[b], acc_sh.at[idxd_v.at[j]],
                              ssem[b]).start(add=True)

    def swt(b):
        pltpu.make_async_copy(rows[b], acc_sh.at[idxd_v.at[0]],
                              ssem[b]).wait()

    for b in range(4):
        gst(b, b)

    def body(k, carry):
        j = k * 4
        for b in range(4):
            gwt(b)
            sst(j + b, b)
        for b in range(4):
            swt(b)
            gst(j + 4 + b, b)
        return carry

    lax.fori_loop(0, NCH // 4 - 1, body, 0)
    for b in range(4):
        gwt(b)
        sst(NCH - 4 + b, b)
    for b in range(4):
        swt(b)
    plsc.subcore_barrier()
    pltpu.sync_copy(acc_sh.at[pl.ds(sid * RPT, RPT)],
                    out_hbm.at[cid, pl.ds(sid * RPT, RPT)])


_seg_call = functools.partial(
    pl.kernel,
    out_type=jax.ShapeDtypeStruct((NC, NA, H), jnp.float32),
    mesh=plsc.VectorSubcoreMesh(core_axis_name="c", subcore_axis_name="s"),
    compiler_params=pltpu.CompilerParams(use_tc_tiling_on_sc=False),
    scratch_types=(
        [pltpu.VMEM_SHARED((NA, H), jnp.float32),
         pltpu.VMEM((NCH, CW), jnp.int32),
         pltpu.VMEM((NCH, CW), jnp.int32)]
        + [pltpu.VMEM((CW, H), jnp.float32)] * 4
        + [pltpu.SemaphoreType.DMA] * 8
    ),
)(_seg_body)


def _full(shape):
    return pl.BlockSpec(shape, lambda: tuple(0 for _ in shape))


def _pre_body(x_ref, w0_ref, b0_ref, w1bd_ref, b1_ref, wmbd_ref, bm_ref,
              hp_ref, m_ref):
    x3 = x_ref[...].reshape(NP, 8, F)
    w0 = w0_ref[...]
    parts = [_dot(x3[:, a, :], w0) for a in range(8)]
    h = _sigbn(jnp.concatenate(parts, axis=1) + b0_ref[...])
    h = _sigbn(_dot(h, w1bd_ref[...]) + b1_ref[...])
    hp_ref[...] = h
    m_ref[...] = _dot(h, wmbd_ref[...]) + bm_ref[...]


def _pre_call(x, w0, b0t, w1bd, b1t, wmbd, bmt):
    return pl.pallas_call(
        _pre_body,
        grid=(),
        in_specs=[_full((N, F)), _full((F, H)), _full((1, PK)),
                  _full((PK, PK)), _full((1, PK)),
                  _full((PK, PK)), _full((1, PK))],
        out_specs=[_full((NP, PK)), _full((NP, PK))],
        out_shape=[jax.ShapeDtypeStruct((NP, PK), jnp.float32),
                   jax.ShapeDtypeStruct((NP, PK), jnp.float32)],
    )(x, w0, b0t, w1bd, b1t, wmbd, bmt)


def _make_mid_body(nz):
    def body(*refs):
        p_ref = refs[0]
        piece_refs = refs[1:1 + nz]
        w_refs = refs[1 + nz:2 + 2 * nz]
        b_ref = refs[2 + 2 * nz]
        z_ref, m_ref = refs[3 + 2 * nz], refs[4 + 2 * nz]
        z = _sigbn(p_ref[0, :NP, :] + p_ref[1, :NP, :])
        z_ref[...] = z
        m = _dot(z, w_refs[0][...]) + b_ref[...]
        for pr, wr in zip(piece_refs, w_refs[1:]):
            m = m + _dot(pr[...], wr[...])
        m_ref[...] = m
    return body


def _mid_call(p, pieces, wbd_slices, bt):
    nz = len(pieces)
    return pl.pallas_call(
        _make_mid_body(nz),
        grid=(),
        in_specs=([_full((NC, NA * H // PK, PK))]
                  + [_full((NP, PK))] * nz
                  + [_full((PK, PK))] * (nz + 1)
                  + [_full((1, PK))]),
        out_specs=[_full((NP, PK)), _full((NP, PK))],
        out_shape=[jax.ShapeDtypeStruct((NP, PK), jnp.float32),
                   jax.ShapeDtypeStruct((NP, PK), jnp.float32)],
    )(p, *pieces, *wbd_slices, bt)


def _final_body(p_ref, z3_ref, z2_ref, z1_ref, hp_ref, bidx_ref,
                wp0_ref, bp0_ref, wp1_ref, bp1_ref, wo_ref, bo_ref, out_ref):
    z4 = _sigbn(p_ref[0, :NP, :] + p_ref[1, :NP, :])
    pieces = [z4, z3_ref[...], z2_ref[...], z1_ref[...], hp_ref[...]]
    bidx = bidx_ref[...]
    iota = lax.broadcasted_iota(jnp.int32, (B, NP), 0)
    g = jnp.zeros((B, 5 * H), jnp.float32)
    for a in range(8):
        oh = (bidx[a, :][None, :] == iota).astype(jnp.float32)
        cols = jnp.concatenate([pc[:, a * H:(a + 1) * H] for pc in pieces],
                               axis=1)
        g = g + _dot(oh, cols)
    g = _sigbn(_dot(g, wp0_ref[...]) + bp0_ref[...])
    g = _sigbn(_dot(g, wp1_ref[...]) + bp1_ref[...])
    out_ref[...] = jax.nn.sigmoid(_dot(g, wo_ref[...]) + bo_ref[...])


def _final_call(p, pieces, bidx_s, wp0, bp0, wp1, bp1, wo, bo):
    return pl.pallas_call(
        _final_body,
        grid=(),
        in_specs=([_full((NC, NA * H // PK, PK))]
                  + [_full((NP, PK))] * 4
                  + [_full((8, NP))]
                  + [_full((5 * H, H)), _full((1, H)),
                     _full((H, H)), _full((1, H)),
                     _full((H, 1)), _full((1, 1))]),
        out_specs=_full((B, 1)),
        out_shape=jax.ShapeDtypeStruct((B, 1), jnp.float32),
    )(p, *pieces, bidx_s, wp0, bp0, wp1, bp1, wo, bo)


def kernel(x, edge_index, batch_idx, W_pre0, b_pre0, W_pre1, b_pre1,
           W_mp0, b_mp0, W_mp1, b_mp1, W_mp2, b_mp2, W_mp3, b_mp3,
           W_post0, b_post0, W_post1, b_post1, W_out, b_out):
    npad = EPAD - E
    pad_i = jnp.arange(npad, dtype=jnp.int32)
    src_r = jnp.concatenate([edge_index[0], (pad_i * 37) % N]) \
        .reshape(NW, NCH, CW)
    dst_r = jnp.concatenate([edge_index[1], N + (pad_i % 16)]) \
        .reshape(NW, NCH, CW)
    zeros = jnp.zeros((NA, H), jnp.float32)
    bidx_s = batch_idx.reshape(NP, 8).T

    def bd(wm):
        return jnp.kron(jnp.eye(8, dtype=jnp.float32), wm)

    def bt(b):
        return jnp.tile(b, 8).reshape(1, PK)

    def bd_slices(wm):
        return [bd(wm[k * H:(k + 1) * H]) for k in range(wm.shape[0] // H)]

    hp, m = _pre_call(x, W_pre0, bt(b_pre0), bd(W_pre1), bt(b_pre1),
                      bd(W_mp0), bt(b_mp0))
    pieces = [hp]
    for wm, bm in ((W_mp1, b_mp1), (W_mp2, b_mp2), (W_mp3, b_mp3)):
        p = _seg_call(m.reshape(N, H), src_r, dst_r, zeros)
        z, m = _mid_call(p.reshape(NC, NA * H // PK, PK), pieces,
                         bd_slices(wm), bt(bm))
        pieces.insert(0, z)
    p = _seg_call(m.reshape(N, H), src_r, dst_r, zeros)
    return _final_call(p.reshape(NC, NA * H // PK, PK), pieces, bidx_s,
                       W_post0, b_post0.reshape(1, H),
                       W_post1, b_post1.reshape(1, H),
                       W_out, b_out.reshape(1, 1))
